# baseline (speedup 1.0000x reference)
import functools

import jax
import jax.numpy as jnp
from jax import lax
from jax.experimental import pallas as pl
from jax.experimental.pallas import tpu as pltpu
from jax.experimental.pallas import tpu_sc as plsc

NC = 2
NS = 16
L = 16
C = 128


def _pre_mm_body(x_ref, w1a_ref, w1b_ref, b1_ref, a_ref, b_ref):
    xb = x_ref[...]
    a_ref[...] = (
        jnp.dot(xb, w1a_ref[...], preferred_element_type=jnp.float32)
        + b1_ref[...]
    )
    b_ref[...] = jnp.dot(xb, w1b_ref[...], preferred_element_type=jnp.float32)


def _pre_mm(x, w1a, w1b, b1):
    n, f = x.shape
    h = w1a.shape[1]
    blk = 400
    grid = (n // blk,)
    return pl.pallas_call(
        _pre_mm_body,
        grid=grid,
        in_specs=[
            pl.BlockSpec((blk, f), lambda i: (i, 0)),
            pl.BlockSpec((f, h), lambda i: (0, 0)),
            pl.BlockSpec((f, h), lambda i: (0, 0)),
            pl.BlockSpec((1, h), lambda i: (0, 0)),
        ],
        out_specs=[
            pl.BlockSpec((blk, h), lambda i: (i, 0)),
            pl.BlockSpec((blk, h), lambda i: (i, 0)),
        ],
        out_shape=[
            jax.ShapeDtypeStruct((n, h), jnp.float32),
            jax.ShapeDtypeStruct((n, h), jnp.float32),
        ],
    )(x, w1a, w1b, b1)


CA = 80


def _masks_sc(a, b, row, col, w2t, b2pad, e, h):
    ew = e // (NC * NS)
    nfull = ew // CA
    npair = nfull // 2
    tail_base = ew - CA
    nj = h // L
    mesh = plsc.VectorSubcoreMesh(core_axis_name="c", subcore_axis_name="s")

    @functools.partial(
        pl.kernel,
        out_type=jax.ShapeDtypeStruct((e,), jnp.float32),
        mesh=mesh,
        scratch_types=[
            pltpu.VMEM((ew,), jnp.int32),
            pltpu.VMEM((ew,), jnp.int32),
            pltpu.VMEM((ew,), jnp.float32),
            pltpu.VMEM((CA, h), jnp.float32),
            pltpu.VMEM((CA, h), jnp.float32),
            pltpu.VMEM((CA, h), jnp.float32),
            pltpu.VMEM((CA, h), jnp.float32),
            pltpu.VMEM((nj, L), jnp.float32),
            pltpu.VMEM((L,), jnp.float32),
            pltpu.SemaphoreType.DMA,
            pltpu.SemaphoreType.DMA,
            pltpu.SemaphoreType.DMA,
            pltpu.SemaphoreType.DMA,
        ],
    )
    def k(a_hbm, b_hbm, row_hbm, col_hbm, w2_hbm, b2_hbm, masks_hbm,
          row_v, col_v, s_v, a0, b0, a1, b1v, w2_v, b2_v,
          sa0, sb0, sa1, sb1):
        wid = lax.axis_index("s") * NC + lax.axis_index("c")
        base0 = wid * ew
        pltpu.sync_copy(w2_hbm, w2_v)
        pltpu.sync_copy(b2_hbm, b2_v)
        pltpu.sync_copy(row_hbm.at[pl.ds(base0, ew)], row_v)
        pltpu.sync_copy(col_hbm.at[pl.ds(base0, ew)], col_v)
        b2reg = b2_v[...]
        lanes = lax.broadcasted_iota(jnp.int32, (L,), 0)

        def start_gather(base, av, bv, sa, sb):
            ca = pltpu.async_copy(a_hbm.at[row_v.at[pl.ds(base, CA)]], av, sa)
            cb = pltpu.async_copy(b_hbm.at[col_v.at[pl.ds(base, CA)]], bv, sb)
            return ca, cb

        w2regs = [w2_v[j] for j in range(nj)]
        perms = [(lanes + sh) % L for sh in (8, 4, 2, 1)]
        zero16 = jnp.zeros((L,), jnp.float32)

        def process(base, av, bv):
            def edge_body(ei, sv):
                accs = [zero16, zero16, zero16, zero16]
                for j in range(nj):
                    va = av[ei, pl.ds(j * L, L)]
                    vb = bv[ei, pl.ds(j * L, L)]
                    hh = jnp.maximum(va + vb, 0.0)
                    accs[j % 4] = accs[j % 4] + hh * w2regs[j]
                sc = (accs[0] + accs[1]) + (accs[2] + accs[3])
                for p in perms:
                    sc = sc + jnp.take(sc, p)
                sv = jnp.where(lanes == (ei & (L - 1)), sc, sv)
                s_v[pl.ds(base + (ei & ~(L - 1)), L)] = sv
                return sv

            lax.fori_loop(0, CA, edge_body, zero16)

            def sig_body(g, _):
                sl = pl.ds(base + g * L, L)
                s_v[sl] = 1.0 / (1.0 + jnp.exp(-(s_v[sl] + b2reg)))
                return 0

            lax.fori_loop(0, CA // L, sig_body, 0)

        start_gather(0, a0, b0, sa0, sb0)

        def pair_body(i, _):
            e0 = pl.multiple_of(i * (2 * CA), 2 * CA)
            c1a, c1b = start_gather(e0 + CA, a1, b1v, sa1, sb1)
            pltpu.make_async_copy(a_hbm.at[row_v.at[pl.ds(e0, CA)]],
                                  a0, sa0).wait()
            pltpu.make_async_copy(b_hbm.at[col_v.at[pl.ds(e0, CA)]],
                                  b0, sb0).wait()
            process(e0, a0, b0)

            @pl.when(i < npair - 1)
            def _():
                start_gather(e0 + 2 * CA, a0, b0, sa0, sb0)

            c1a.wait()
            c1b.wait()
            process(e0 + CA, a1, b1v)
            return 0

        lax.fori_loop(0, npair, pair_body, 0)
        ca, cb = start_gather(tail_base, a0, b0, sa0, sb0)
        ca.wait()
        cb.wait()
        process(tail_base, a0, b0)
        pltpu.sync_copy(s_v, masks_hbm.at[pl.ds(base0, ew)])

    return k(a, b, row, col, w2t, b2pad)


def _agg_sc(x16, row, col, masks, n, e, f):
    nh = n // NC
    rs = (-(-nh // NS) + 7) // 8 * 8
    rs_last = nh - (NS - 1) * rs
    BB = 3200
    CG = 400
    ncpb = BB // CG
    nblk = e // BB
    npair = nblk // 2
    assert npair * 2 * BB == e and ncpb * CG == BB
    mesh = plsc.VectorSubcoreMesh(core_axis_name="c", subcore_axis_name="s")

    @functools.partial(
        pl.kernel,
        out_type=[
            jax.ShapeDtypeStruct((n, f), jnp.float32),
            jax.ShapeDtypeStruct((n, L), jnp.float32),
        ],
        mesh=mesh,
        compiler_params=pltpu.CompilerParams(
            needs_layout_passes=False, use_tc_tiling_on_sc=False),
        scratch_types=[
            pltpu.VMEM((BB,), jnp.int32),
            pltpu.VMEM((BB,), jnp.int32),
            pltpu.VMEM((BB,), jnp.float32),
            pltpu.VMEM((BB,), jnp.int32),
            pltpu.VMEM((BB,), jnp.int32),
            pltpu.VMEM((BB,), jnp.float32),
            pltpu.VMEM((BB,), jnp.int32),
            pltpu.VMEM((CG, L), jnp.float32),
            pltpu.VMEM((CG, L), jnp.float32),
            pltpu.VMEM((nh, L), jnp.float32),
            pltpu.VMEM((rs + 1, L), jnp.float32),
            pltpu.SemaphoreType.DMA,
            pltpu.SemaphoreType.DMA,
            pltpu.SemaphoreType.DMA,
            pltpu.SemaphoreType.DMA,
        ],
    )
    def k(x_hbm, row_hbm, col_hbm, m_hbm, agg_hbm, norm_hbm,
          row0, col0, m0, row1, col1, m1, idxg, xv0, xv1, acc, nacc,
          si0, si1, sx0, sx1):
        cid = lax.axis_index("c")
        sid = lax.axis_index("s")
        half_off = cid * nh
        norm_off = sid * rs
        norm_cnt = jnp.where(sid == NS - 1, rs_last, rs)
        zero16 = jnp.zeros((L,), jnp.float32)
        zeros_i = jnp.zeros((L,), jnp.int32)
        lanes = lax.broadcasted_iota(jnp.int32, (L,), 0)

        def z_body(i, _):
            acc[i, pl.ds(0, L)] = zero16
            return 0

        lax.fori_loop(0, nh, z_body, 0)

        def zn_body(i, _):
            nacc[i, pl.ds(0, L)] = zero16
            return 0

        lax.fori_loop(0, rs + 1, zn_body, 0)

        def start_idx(base, rv, cv, mv, sem):
            pltpu.async_copy(row_hbm.at[pl.ds(base, BB)], rv, sem)
            pltpu.async_copy(col_hbm.at[pl.ds(base, BB)], cv, sem)
            pltpu.async_copy(m_hbm.at[pl.ds(base, BB)], mv, sem)

        def drain_idx(rv, cv, mv, sem):
            pltpu.make_async_copy(row_hbm.at[pl.ds(0, BB)], rv, sem).wait()
            pltpu.make_async_copy(col_hbm.at[pl.ds(0, BB)], cv, sem).wait()
            pltpu.make_async_copy(m_hbm.at[pl.ds(0, BB)], mv, sem).wait()

        def do_block(rv, cv, mv):
            def gi_body(g, _):
                sl = pl.ds(g * L, L)
                idxg[sl] = cv[sl] * L + sid
                return 0

            lax.fori_loop(0, BB // L, gi_body, 0)

            def start_g(c, xv, sem):
                return pltpu.async_copy(
                    x_hbm.at[idxg.at[pl.ds(c * CG, CG)]], xv, sem)

            def process(c, xv):
                off = c * CG

                def group_body(g, _):
                    sl = pl.ds(off + g * L, L)
                    rl = rv[sl] - half_off
                    ind = (rl >= 0) & (rl < nh)
                    rlc = jnp.where(ind, rl, 0)
                    me = jnp.where(ind, mv[sl], 0.0)
                    rl2 = rl - norm_off
                    ok2 = ind & (rl2 >= 0) & (rl2 < norm_cnt)
                    rn = jnp.where(ok2, rl2, rs)
                    depth = 6

                    def mk(kk):
                        v = xv[g * L + kk, pl.ds(0, L)]
                        msp = jnp.take(me, jnp.full((L,), kk, jnp.int32))
                        return rlc[kk], v * msp

                    pipe = [mk(kk) for kk in range(depth)]
                    for kk in range(L):
                        if kk + depth < L:
                            pipe.append(mk(kk + depth))
                        r, v = pipe[kk]
                        plsc.addupdate(acc.at[r], v)
                    plsc.addupdate_scatter(nacc, [rn, zeros_i], me)
                    return 0

                lax.fori_loop(0, CG // L, group_body, 0)

            descs = [start_g(0, xv0, sx0)]
            for c in range(ncpb):
                if c + 1 < ncpb:
                    descs.append(
                        start_g(c + 1, (xv0, xv1)[(c + 1) % 2],
                                (sx0, sx1)[(c + 1) % 2]))
                descs[c].wait()
                process(c, (xv0, xv1)[c % 2])

        start_idx(0, row0, col0, m0, si0)

        def pair_body(i, _):
            e0 = pl.multiple_of(i * (2 * BB), 2 * BB)
            start_idx(e0 + BB, row1, col1, m1, si1)
            drain_idx(row0, col0, m0, si0)
            do_block(row0, col0, m0)

            @pl.when(i < npair - 1)
            def _():
                start_idx(e0 + 2 * BB, row0, col0, m0, si0)

            drain_idx(row1, col1, m1, si1)
            do_block(row1, col1, m1)
            return 0

        lax.fori_loop(0, npair, pair_body, 0)

        pltpu.sync_copy(
            acc, agg_hbm.at[pl.ds(half_off, nh), pl.ds(sid * L, L)])

        @pl.when(sid < NS - 1)
        def _():
            pltpu.sync_copy(
                nacc.at[pl.ds(0, rs)],
                norm_hbm.at[pl.ds(half_off + norm_off, rs)])

        @pl.when(sid == NS - 1)
        def _():
            pltpu.sync_copy(
                nacc.at[pl.ds(0, rs_last)],
                norm_hbm.at[pl.ds(half_off + norm_off, rs_last)])

    return k(x16, row, col, masks)


def _out_mm_body(agg_ref, norm_ref, x_ref, wt_ref, o_ref):
    normd = agg_ref[...] / (norm_ref[:, 0:1] + 1e-9)
    comb = normd + x_ref[...]
    o_ref[...] = jnp.maximum(
        jnp.dot(comb, wt_ref[...], preferred_element_type=jnp.float32), 0.0
    )


def _out_mm(agg, norm, x, wlin_t):
    n, f = x.shape
    out = wlin_t.shape[1]
    blk = 400
    grid = (n // blk,)
    return pl.pallas_call(
        _out_mm_body,
        grid=grid,
        in_specs=[
            pl.BlockSpec((blk, f), lambda i: (i, 0)),
            pl.BlockSpec((blk, L), lambda i: (i, 0)),
            pl.BlockSpec((blk, f), lambda i: (i, 0)),
            pl.BlockSpec((f, out), lambda i: (0, 0)),
        ],
        out_specs=pl.BlockSpec((blk, out), lambda i: (i, 0)),
        out_shape=jax.ShapeDtypeStruct((n, out), jnp.float32),
    )(agg, norm, x, wlin_t)


def kernel(x, edge_index, W1, b1, W2, b2, Wlin):
    n, f = x.shape
    h = W1.shape[1]
    e = edge_index.shape[1]
    row = edge_index[0]
    col = edge_index[1]
    w1a = W1[:f]
    w1b = W1[f:]
    b1r = b1.reshape(1, h)
    w2t = W2[:, 0].reshape(h // L, L)
    b2pad = jnp.broadcast_to(b2, (L,))
    wlin_t = Wlin.T

    a, b = _pre_mm(x, w1a, w1b, b1r)
    masks = _masks_sc(a, b, row, col, w2t, b2pad, e, h)
    x16 = x.reshape(n * (f // L), L)
    agg, norm = _agg_sc(x16, row, col, masks, n, e, f)
    out = _out_mm(agg, norm, x, wlin_t)
    return (out, masks)


# speedup vs baseline: 1.0802x; 1.0802x over previous
import functools

import jax
import jax.numpy as jnp
from jax import lax
from jax.experimental import pallas as pl
from jax.experimental.pallas import tpu as pltpu
from jax.experimental.pallas import tpu_sc as plsc

NC = 2
NS = 16
L = 16
C = 128


def _pre_mm_body(x_ref, w1a_ref, w1b_ref, b1_ref, a_ref, b_ref):
    xb = x_ref[...]
    a_ref[...] = (
        jnp.dot(xb, w1a_ref[...], preferred_element_type=jnp.float32)
        + b1_ref[...]
    )
    b_ref[...] = jnp.dot(xb, w1b_ref[...], preferred_element_type=jnp.float32)


def _pre_mm(x, w1a, w1b, b1):
    n, f = x.shape
    h = w1a.shape[1]
    blk = 400
    grid = (n // blk,)
    return pl.pallas_call(
        _pre_mm_body,
        grid=grid,
        in_specs=[
            pl.BlockSpec((blk, f), lambda i: (i, 0)),
            pl.BlockSpec((f, h), lambda i: (0, 0)),
            pl.BlockSpec((f, h), lambda i: (0, 0)),
            pl.BlockSpec((1, h), lambda i: (0, 0)),
        ],
        out_specs=[
            pl.BlockSpec((blk, h), lambda i: (i, 0)),
            pl.BlockSpec((blk, h), lambda i: (i, 0)),
        ],
        out_shape=[
            jax.ShapeDtypeStruct((n, h), jnp.float32),
            jax.ShapeDtypeStruct((n, h), jnp.float32),
        ],
    )(x, w1a, w1b, b1)


CA = 80


def _masks_sc(a, b, row, col, w2t, b2pad, e, h):
    ew = e // (NC * NS)
    nfull = ew // CA
    npair = nfull // 2
    tail_base = ew - CA
    nj = h // L
    mesh = plsc.VectorSubcoreMesh(core_axis_name="c", subcore_axis_name="s")

    @functools.partial(
        pl.kernel,
        out_type=jax.ShapeDtypeStruct((e,), jnp.float32),
        mesh=mesh,
        scratch_types=[
            pltpu.VMEM((ew,), jnp.int32),
            pltpu.VMEM((ew,), jnp.int32),
            pltpu.VMEM((ew,), jnp.float32),
            pltpu.VMEM((CA, h), jnp.float32),
            pltpu.VMEM((CA, h), jnp.float32),
            pltpu.VMEM((CA, h), jnp.float32),
            pltpu.VMEM((CA, h), jnp.float32),
            pltpu.VMEM((nj, L), jnp.float32),
            pltpu.VMEM((L,), jnp.float32),
            pltpu.SemaphoreType.DMA,
            pltpu.SemaphoreType.DMA,
            pltpu.SemaphoreType.DMA,
            pltpu.SemaphoreType.DMA,
        ],
    )
    def k(a_hbm, b_hbm, row_hbm, col_hbm, w2_hbm, b2_hbm, masks_hbm,
          row_v, col_v, s_v, a0, b0, a1, b1v, w2_v, b2_v,
          sa0, sb0, sa1, sb1):
        wid = lax.axis_index("s") * NC + lax.axis_index("c")
        base0 = wid * ew
        pltpu.sync_copy(w2_hbm, w2_v)
        pltpu.sync_copy(b2_hbm, b2_v)
        pltpu.sync_copy(row_hbm.at[pl.ds(base0, ew)], row_v)
        pltpu.sync_copy(col_hbm.at[pl.ds(base0, ew)], col_v)
        b2reg = b2_v[...]
        lanes = lax.broadcasted_iota(jnp.int32, (L,), 0)

        def start_gather(base, av, bv, sa, sb):
            ca = pltpu.async_copy(a_hbm.at[row_v.at[pl.ds(base, CA)]], av, sa)
            cb = pltpu.async_copy(b_hbm.at[col_v.at[pl.ds(base, CA)]], bv, sb)
            return ca, cb

        w2regs = [w2_v[j] for j in range(nj)]
        perms = [(lanes + sh) % L for sh in (8, 4, 2, 1)]
        zero16 = jnp.zeros((L,), jnp.float32)

        def process(base, av, bv):
            def edge_body(eo, sv):
                scs = []
                for kk in range(2):
                    ei = eo * 2 + kk
                    accs = [zero16, zero16, zero16, zero16]
                    for j in range(nj):
                        va = av[ei, pl.ds(j * L, L)]
                        vb = bv[ei, pl.ds(j * L, L)]
                        hh = jnp.maximum(va + vb, 0.0)
                        accs[j % 4] = accs[j % 4] + hh * w2regs[j]
                    scs.append((accs[0] + accs[1]) + (accs[2] + accs[3]))
                for p in perms:
                    scs = [sc + jnp.take(sc, p) for sc in scs]
                e0_ = eo * 2
                sv = jnp.where(lanes == (e0_ & (L - 1)), scs[0], sv)
                sv = jnp.where(lanes == ((e0_ + 1) & (L - 1)), scs[1], sv)
                s_v[pl.ds(base + (e0_ & ~(L - 1)), L)] = sv
                return sv

            lax.fori_loop(0, CA // 2, edge_body, zero16)

            def sig_body(g, _):
                sl = pl.ds(base + g * L, L)
                s_v[sl] = 1.0 / (1.0 + jnp.exp(-(s_v[sl] + b2reg)))
                return 0

            lax.fori_loop(0, CA // L, sig_body, 0)

        start_gather(0, a0, b0, sa0, sb0)

        def pair_body(i, _):
            e0 = pl.multiple_of(i * (2 * CA), 2 * CA)
            c1a, c1b = start_gather(e0 + CA, a1, b1v, sa1, sb1)
            pltpu.make_async_copy(a_hbm.at[row_v.at[pl.ds(e0, CA)]],
                                  a0, sa0).wait()
            pltpu.make_async_copy(b_hbm.at[col_v.at[pl.ds(e0, CA)]],
                                  b0, sb0).wait()
            process(e0, a0, b0)

            @pl.when(i < npair - 1)
            def _():
                start_gather(e0 + 2 * CA, a0, b0, sa0, sb0)

            c1a.wait()
            c1b.wait()
            process(e0 + CA, a1, b1v)
            return 0

        lax.fori_loop(0, npair, pair_body, 0)
        ca, cb = start_gather(tail_base, a0, b0, sa0, sb0)
        ca.wait()
        cb.wait()
        process(tail_base, a0, b0)
        pltpu.sync_copy(s_v, masks_hbm.at[pl.ds(base0, ew)])

    return k(a, b, row, col, w2t, b2pad)


def _agg_sc(x16, row, col, masks, n, e, f):
    nh = n // NC
    rs = (-(-nh // NS) + 7) // 8 * 8
    rs_last = nh - (NS - 1) * rs
    BB = 3200
    CG = 400
    ncpb = BB // CG
    nblk = e // BB
    npair = nblk // 2
    assert npair * 2 * BB == e and ncpb * CG == BB
    mesh = plsc.VectorSubcoreMesh(core_axis_name="c", subcore_axis_name="s")

    @functools.partial(
        pl.kernel,
        out_type=[
            jax.ShapeDtypeStruct((n, f), jnp.float32),
            jax.ShapeDtypeStruct((n, L), jnp.float32),
        ],
        mesh=mesh,
        compiler_params=pltpu.CompilerParams(
            needs_layout_passes=False, use_tc_tiling_on_sc=False),
        scratch_types=[
            pltpu.VMEM((BB,), jnp.int32),
            pltpu.VMEM((BB,), jnp.int32),
            pltpu.VMEM((BB,), jnp.float32),
            pltpu.VMEM((BB,), jnp.int32),
            pltpu.VMEM((BB,), jnp.int32),
            pltpu.VMEM((BB,), jnp.float32),
            pltpu.VMEM((BB,), jnp.int32),
            pltpu.VMEM((CG, L), jnp.float32),
            pltpu.VMEM((CG, L), jnp.float32),
            pltpu.VMEM((nh, L), jnp.float32),
            pltpu.VMEM((rs + 1, L), jnp.float32),
            pltpu.SemaphoreType.DMA,
            pltpu.SemaphoreType.DMA,
            pltpu.SemaphoreType.DMA,
            pltpu.SemaphoreType.DMA,
        ],
    )
    def k(x_hbm, row_hbm, col_hbm, m_hbm, agg_hbm, norm_hbm,
          row0, col0, m0, row1, col1, m1, idxg, xv0, xv1, acc, nacc,
          si0, si1, sx0, sx1):
        cid = lax.axis_index("c")
        sid = lax.axis_index("s")
        half_off = cid * nh
        norm_off = sid * rs
        norm_cnt = jnp.where(sid == NS - 1, rs_last, rs)
        zero16 = jnp.zeros((L,), jnp.float32)
        zeros_i = jnp.zeros((L,), jnp.int32)
        lanes = lax.broadcasted_iota(jnp.int32, (L,), 0)

        def z_body(i, _):
            acc[i, pl.ds(0, L)] = zero16
            return 0

        lax.fori_loop(0, nh, z_body, 0)

        def zn_body(i, _):
            nacc[i, pl.ds(0, L)] = zero16
            return 0

        lax.fori_loop(0, rs + 1, zn_body, 0)

        def start_idx(base, rv, cv, mv, sem):
            pltpu.async_copy(row_hbm.at[pl.ds(base, BB)], rv, sem)
            pltpu.async_copy(col_hbm.at[pl.ds(base, BB)], cv, sem)
            pltpu.async_copy(m_hbm.at[pl.ds(base, BB)], mv, sem)

        def drain_idx(rv, cv, mv, sem):
            pltpu.make_async_copy(row_hbm.at[pl.ds(0, BB)], rv, sem).wait()
            pltpu.make_async_copy(col_hbm.at[pl.ds(0, BB)], cv, sem).wait()
            pltpu.make_async_copy(m_hbm.at[pl.ds(0, BB)], mv, sem).wait()

        def do_block(rv, cv, mv):
            def gi_body(g, _):
                sl = pl.ds(g * L, L)
                idxg[sl] = cv[sl] * L + sid
                return 0

            lax.fori_loop(0, BB // L, gi_body, 0)

            def start_g(c, xv, sem):
                return pltpu.async_copy(
                    x_hbm.at[idxg.at[pl.ds(c * CG, CG)]], xv, sem)

            def process(c, xv):
                off = c * CG

                def group_body(g, _):
                    sl = pl.ds(off + g * L, L)
                    rl = rv[sl] - half_off
                    ind = (rl >= 0) & (rl < nh)
                    rlc = jnp.where(ind, rl, 0)
                    me = jnp.where(ind, mv[sl], 0.0)
                    rl2 = rl - norm_off
                    ok2 = ind & (rl2 >= 0) & (rl2 < norm_cnt)
                    rn = jnp.where(ok2, rl2, rs)
                    rrs = [rlc[kk] for kk in range(L)]
                    vvs = []
                    for kk in range(L):
                        ei = g * L + kk
                        v = xv[ei, pl.ds(0, L)]
                        msp = jnp.take(me, jnp.full((L,), kk, jnp.int32))
                        vvs.append(v * msp)
                    for kk in range(L):
                        plsc.addupdate(acc.at[rrs[kk]], vvs[kk])
                    plsc.addupdate_scatter(nacc, [rn, zeros_i], me)
                    return 0

                lax.fori_loop(0, CG // L, group_body, 0)

            descs = [start_g(0, xv0, sx0)]
            for c in range(ncpb):
                if c + 1 < ncpb:
                    descs.append(
                        start_g(c + 1, (xv0, xv1)[(c + 1) % 2],
                                (sx0, sx1)[(c + 1) % 2]))
                descs[c].wait()
                process(c, (xv0, xv1)[c % 2])

        start_idx(0, row0, col0, m0, si0)

        def pair_body(i, _):
            e0 = pl.multiple_of(i * (2 * BB), 2 * BB)
            start_idx(e0 + BB, row1, col1, m1, si1)
            drain_idx(row0, col0, m0, si0)
            do_block(row0, col0, m0)

            @pl.when(i < npair - 1)
            def _():
                start_idx(e0 + 2 * BB, row0, col0, m0, si0)

            drain_idx(row1, col1, m1, si1)
            do_block(row1, col1, m1)
            return 0

        lax.fori_loop(0, npair, pair_body, 0)

        pltpu.sync_copy(
            acc, agg_hbm.at[pl.ds(half_off, nh), pl.ds(sid * L, L)])

        @pl.when(sid < NS - 1)
        def _():
            pltpu.sync_copy(
                nacc.at[pl.ds(0, rs)],
                norm_hbm.at[pl.ds(half_off + norm_off, rs)])

        @pl.when(sid == NS - 1)
        def _():
            pltpu.sync_copy(
                nacc.at[pl.ds(0, rs_last)],
                norm_hbm.at[pl.ds(half_off + norm_off, rs_last)])

    return k(x16, row, col, masks)


def _out_mm_body(agg_ref, norm_ref, x_ref, wt_ref, o_ref):
    normd = agg_ref[...] / (norm_ref[:, 0:1] + 1e-9)
    comb = normd + x_ref[...]
    o_ref[...] = jnp.maximum(
        jnp.dot(comb, wt_ref[...], preferred_element_type=jnp.float32), 0.0
    )


def _out_mm(agg, norm, x, wlin_t):
    n, f = x.shape
    out = wlin_t.shape[1]
    blk = 400
    grid = (n // blk,)
    return pl.pallas_call(
        _out_mm_body,
        grid=grid,
        in_specs=[
            pl.BlockSpec((blk, f), lambda i: (i, 0)),
            pl.BlockSpec((blk, L), lambda i: (i, 0)),
            pl.BlockSpec((blk, f), lambda i: (i, 0)),
            pl.BlockSpec((f, out), lambda i: (0, 0)),
        ],
        out_specs=pl.BlockSpec((blk, out), lambda i: (i, 0)),
        out_shape=jax.ShapeDtypeStruct((n, out), jnp.float32),
    )(agg, norm, x, wlin_t)


def kernel(x, edge_index, W1, b1, W2, b2, Wlin):
    n, f = x.shape
    h = W1.shape[1]
    e = edge_index.shape[1]
    row = edge_index[0]
    col = edge_index[1]
    w1a = W1[:f]
    w1b = W1[f:]
    b1r = b1.reshape(1, h)
    w2t = W2[:, 0].reshape(h // L, L)
    b2pad = jnp.broadcast_to(b2, (L,))
    wlin_t = Wlin.T

    a, b = _pre_mm(x, w1a, w1b, b1r)
    masks = _masks_sc(a, b, row, col, w2t, b2pad, e, h)
    x16 = x.reshape(n * (f // L), L)
    agg, norm = _agg_sc(x16, row, col, masks, n, e, f)
    out = _out_mm(agg, norm, x, wlin_t)
    return (out, masks)
